# baseline (speedup 1.0000x reference)
import functools

import jax
import jax.numpy as jnp
from jax import lax
from jax.experimental import pallas as pl
from jax.experimental.pallas import tpu as pltpu
from jax.experimental.pallas import tpu_sc as plsc

NC = 2
NS = 16
NW = NC * NS
LANES = 16

C_SEQ = 2
SUB = 100


def _positional_encoding(seq_len: int, d_model: int) -> jax.Array:
    pos = jnp.arange(seq_len, dtype=jnp.float32)[:, None]
    _2i = jnp.arange(0, d_model, 2, dtype=jnp.float32)
    enc = jnp.zeros((seq_len, d_model), dtype=jnp.float32)
    enc = enc.at[:, 0::2].set(jnp.sin(pos / (10000.0 ** (_2i / d_model))))
    enc = enc.at[:, 1::2].set(jnp.cos(pos / (10000.0 ** (_2i / d_model))))
    return enc


@functools.partial(jax.jit, static_argnames=("B", "S", "D"))
def _embed_sc(idx_e, idx_o, table, enc2, *, B, S, D):
    N = B * S
    R = C_SEQ * S
    Q = R * D // 128
    KS = R // 2 // SUB
    rows_per_w = N // NW
    qrows_per_w = rows_per_w * D // 128
    G = rows_per_w // R
    srows_per_w = rows_per_w // 2 // SUB

    mesh = plsc.VectorSubcoreMesh(core_axis_name="c", subcore_axis_name="s")

    @functools.partial(
        pl.kernel,
        mesh=mesh,
        compiler_params=pltpu.CompilerParams(use_tc_tiling_on_sc=False),
        out_type=jax.ShapeDtypeStruct((N * D // 128, 128), jnp.float32),
        scratch_types=[
            pltpu.VMEM((KS, SUB), jnp.int32),
            pltpu.VMEM((KS, SUB), jnp.int32),
            pltpu.VMEM((R, D), jnp.float32),
            pltpu.VMEM((Q, 128), jnp.float32),
            pltpu.VMEM((S // 2, 128), jnp.float32),
            pltpu.SemaphoreType.DMA,
        ],
    )
    def body(ie_hbm, io_hbm, table_hbm, enc_hbm, out_hbm,
             ie_v, io_v, gbuf_v, buf_v, enc_v, sem):
        wid = lax.axis_index("s") * NC + lax.axis_index("c")
        pltpu.sync_copy(enc_hbm, enc_v)

        def chunk(g, carry):
            qrow0 = wid * qrows_per_w + g * Q
            srow0 = wid * srows_per_w + g * KS
            pltpu.sync_copy(ie_hbm.at[pl.ds(srow0, KS), :], ie_v)
            pltpu.sync_copy(io_hbm.at[pl.ds(srow0, KS), :], io_v)
            cps = []
            for k in range(KS):
                cps.append(pltpu.async_copy(
                    table_hbm.at[ie_v.at[k]],
                    gbuf_v.at[pl.ds(k * SUB, SUB), :],
                    sem,
                ))
                cps.append(pltpu.async_copy(
                    table_hbm.at[io_v.at[k]],
                    gbuf_v.at[pl.ds((KS + k) * SUB, SUB), :],
                    sem,
                ))
            for cp in cps:
                cp.wait()

            def add_row(s2, c2):
                for col in range(128 // LANES):
                    sl = pl.ds(col * LANES, LANES)
                    dsl = pl.ds((col % (D // LANES)) * LANES, LANES)
                    half = (col // (D // LANES)) * (R // 2)
                    e = enc_v[s2, sl]
                    for c in range(C_SEQ):
                        q = c * (S // 2) + s2
                        buf_v[q, sl] = gbuf_v[half + q, dsl] + e
                return c2

            lax.fori_loop(0, S // 2, add_row, 0)
            pltpu.sync_copy(buf_v, out_hbm.at[pl.ds(qrow0, Q), :])
            return carry

        lax.fori_loop(0, G, chunk, 0)

    return body(idx_e, idx_o, table, enc2)


def kernel(x, table):
    B, S = x.shape
    _, D = table.shape
    N = B * S
    xr = x.reshape(N // 2, 2)
    idx_e = xr[:, 0].reshape(N // 2 // SUB, SUB)
    idx_o = xr[:, 1].reshape(N // 2 // SUB, SUB)
    enc2 = _positional_encoding(S, D).reshape(S // 2, 2 * D)
    out = _embed_sc(idx_e, idx_o, table, enc2, B=B, S=S, D=D)
    return out.reshape(B, S, D)


# speedup vs baseline: 1.8056x; 1.8056x over previous
import functools

import jax
import jax.numpy as jnp
from jax import lax
from jax.experimental import pallas as pl
from jax.experimental.pallas import tpu as pltpu
from jax.experimental.pallas import tpu_sc as plsc

NC = 2
NS = 16
NW = NC * NS
LANES = 16

C_SEQ = 2
SUB = 100


def _positional_encoding(seq_len: int, d_model: int) -> jax.Array:
    pos = jnp.arange(seq_len, dtype=jnp.float32)[:, None]
    _2i = jnp.arange(0, d_model, 2, dtype=jnp.float32)
    enc = jnp.zeros((seq_len, d_model), dtype=jnp.float32)
    enc = enc.at[:, 0::2].set(jnp.sin(pos / (10000.0 ** (_2i / d_model))))
    enc = enc.at[:, 1::2].set(jnp.cos(pos / (10000.0 ** (_2i / d_model))))
    return enc


@functools.partial(jax.jit, static_argnames=("B", "S", "D"))
def _embed_sc(idx2d, table, enc, *, B, S, D):
    N = B * S
    R = C_SEQ * S
    KSUB = R // SUB
    rows_per_w = N // NW
    G = rows_per_w // R
    srows_per_w = rows_per_w // SUB

    mesh = plsc.VectorSubcoreMesh(core_axis_name="c", subcore_axis_name="s")

    @functools.partial(
        pl.kernel,
        mesh=mesh,
        compiler_params=pltpu.CompilerParams(use_tc_tiling_on_sc=False),
        out_type=jax.ShapeDtypeStruct((N * D,), jnp.float32),
        scratch_types=[
            pltpu.VMEM((KSUB, SUB), jnp.int32),
            pltpu.VMEM((R, D), jnp.float32),
            pltpu.VMEM((R * D,), jnp.float32),
            pltpu.VMEM((S, D), jnp.float32),
            pltpu.SemaphoreType.DMA,
        ],
    )
    def body(idx_hbm, table_hbm, enc_hbm, out_hbm, idx_v, gbuf_v, obuf_v,
             enc_v, sem):
        wid = lax.axis_index("s") * NC + lax.axis_index("c")
        pltpu.sync_copy(enc_hbm, enc_v)

        def chunk(g, carry):
            row0 = wid * rows_per_w + g * R
            srow0 = wid * srows_per_w + g * KSUB
            pltpu.sync_copy(idx_hbm.at[pl.ds(srow0, KSUB), :], idx_v)
            cps = [
                pltpu.async_copy(
                    table_hbm.at[idx_v.at[k]],
                    gbuf_v.at[pl.ds(k * SUB, SUB), :],
                    sem,
                )
                for k in range(KSUB)
            ]
            for cp in cps:
                cp.wait()

            def add_row(s, c2):
                for d in range(D // LANES):
                    sl = pl.ds(d * LANES, LANES)
                    e = enc_v[s, sl]
                    for c in range(C_SEQ):
                        r = c * S + s
                        obuf_v[pl.ds(r * D + d * LANES, LANES)] = (
                            gbuf_v[r, sl] + e)
                return c2

            lax.fori_loop(0, S, add_row, 0)
            pltpu.sync_copy(obuf_v, out_hbm.at[pl.ds(row0 * D, R * D)])
            return carry

        lax.fori_loop(0, G, chunk, 0)

    return body(idx2d, table, enc)


def kernel(x, table):
    B, S = x.shape
    _, D = table.shape
    idx2d = x.reshape(B * S // SUB, SUB)
    enc = _positional_encoding(S, D)
    out = _embed_sc(idx2d, table, enc, B=B, S=S, D=D)
    return out.reshape(B, S, D)
